# Initial kernel scaffold; baseline (speedup 1.0000x reference)
import jax
import jax.numpy as jnp
from jax.experimental import pallas as pl


def kernel(x, edge_index, W1, b1, beta2, W2, b2):
    raise NotImplementedError("write your pallas kernel here")


# speedup vs baseline: 5.6373x; 5.6373x over previous
import functools

import jax
import jax.numpy as jnp
from jax import lax
from jax.experimental import pallas as pl
from jax.experimental.pallas import tpu as pltpu
from jax.experimental.pallas import tpu_sc as plsc

N = 10000
E = 320000
D = 128
NCLS = 64
NC, NS, L = 2, 16, 16
NW = NC * NS
CH = 64
NCHUNK = 160
EPT = CH * NCHUNK
EPAD = NW * EPT
NPAD = 10240
RPT = NPAD // NS
BR = 128


def _lin1_body(x_ref, w_ref, b_ref, hn_ref, nrm_ref):
    h = jnp.dot(x_ref[...], w_ref[...], preferred_element_type=jnp.float32)
    h = jnp.maximum(h + b_ref[...], 0.0)
    ss = jnp.sum(h * h, axis=1) + 1e-12
    hn_ref[...] = h * lax.rsqrt(ss)[:, None]
    nrm_ref[...] = jnp.sqrt(ss)[None, None, :]


def _lin1(xp, W1, b1r):
    return pl.pallas_call(
        _lin1_body,
        grid=(NPAD // BR,),
        in_specs=[
            pl.BlockSpec((BR, D), lambda i: (i, 0)),
            pl.BlockSpec((D, D), lambda i: (0, 0)),
            pl.BlockSpec((1, D), lambda i: (0, 0)),
        ],
        out_specs=[
            pl.BlockSpec((BR, D), lambda i: (i, 0)),
            pl.BlockSpec((1, 1, BR), lambda i: (i, 0, 0)),
        ],
        out_shape=[
            jax.ShapeDtypeStruct((NPAD, D), jnp.float32),
            jax.ShapeDtypeStruct((NPAD // BR, 1, BR), jnp.float32),
        ],
    )(xp, W1, b1r)


def _comb_body(agg_ref, den_ref, hn_ref, nrm_ref):
    den = den_ref[0, 0, 0, :] + den_ref[1, 0, 0, :] + 1e-16
    h = (agg_ref[0] + agg_ref[1]) / den[:, None]
    ss = jnp.sum(h * h, axis=1) + 1e-12
    hn_ref[...] = h * lax.rsqrt(ss)[:, None]
    nrm_ref[...] = jnp.sqrt(ss)[None, None, :]


def _comb(aggp, denp):
    return pl.pallas_call(
        _comb_body,
        grid=(NPAD // BR,),
        in_specs=[
            pl.BlockSpec((NC, BR, D), lambda i: (0, i, 0)),
            pl.BlockSpec((NC, 1, 1, BR), lambda i: (0, i, 0, 0)),
        ],
        out_specs=[
            pl.BlockSpec((BR, D), lambda i: (i, 0)),
            pl.BlockSpec((1, 1, BR), lambda i: (i, 0, 0)),
        ],
        out_shape=[
            jax.ShapeDtypeStruct((NPAD, D), jnp.float32),
            jax.ShapeDtypeStruct((NPAD // BR, 1, BR), jnp.float32),
        ],
    )(aggp, denp)


def _fin_body(agg_ref, den_ref, w_ref, b_ref, o_ref):
    den = den_ref[0, 0, 0, :] + den_ref[1, 0, 0, :] + 1e-16
    h = (agg_ref[0] + agg_ref[1]) / den[:, None]
    o_ref[...] = (
        jnp.dot(h, w_ref[...], preferred_element_type=jnp.float32) + b_ref[...]
    )


def _fin(aggp, denp, W2, b2r):
    return pl.pallas_call(
        _fin_body,
        grid=(NPAD // BR,),
        in_specs=[
            pl.BlockSpec((NC, BR, D), lambda i: (0, i, 0)),
            pl.BlockSpec((NC, 1, 1, BR), lambda i: (0, i, 0, 0)),
            pl.BlockSpec((D, NCLS), lambda i: (0, 0)),
            pl.BlockSpec((1, NCLS), lambda i: (0, 0)),
        ],
        out_specs=pl.BlockSpec((BR, NCLS), lambda i: (i, 0)),
        out_shape=jax.ShapeDtypeStruct((NPAD, NCLS), jnp.float32),
    )(aggp, denp, W2, b2r)


def _prop_sc(hn_pad, nrm_pad, beta_vec, srcp, dstp):
    mesh = plsc.VectorSubcoreMesh(
        core_axis_name="c", subcore_axis_name="s", num_cores=NC, num_subcores=NS
    )

    @functools.partial(
        pl.kernel,
        out_type=(
            jax.ShapeDtypeStruct((NC * NPAD, D), jnp.float32),
            jax.ShapeDtypeStruct((NC * NPAD,), jnp.float32),
        ),
        mesh=mesh,
        compiler_params=pltpu.CompilerParams(needs_layout_passes=False),
        scratch_types=(
            pltpu.VMEM((CH,), jnp.int32),
            pltpu.VMEM((CH,), jnp.int32),
            pltpu.VMEM((1, CH), jnp.int32),
            pltpu.VMEM((1, CH), jnp.int32),
            pltpu.VMEM((1, CH), jnp.int32),
            pltpu.VMEM((1, CH), jnp.int32),
            pltpu.VMEM((CH, D), jnp.float32),
            pltpu.VMEM((CH, D), jnp.float32),
            pltpu.VMEM((CH, D), jnp.float32),
            pltpu.VMEM((CH, D), jnp.float32),
            pltpu.VMEM((CH,), jnp.float32),
            pltpu.VMEM((CH,), jnp.float32),
            pltpu.VMEM((RPT,), jnp.float32),
            pltpu.VMEM((L * (L + 1),), jnp.float32),
            pltpu.VMEM((NPAD,), jnp.float32),
            pltpu.VMEM((L,), jnp.float32),
            pltpu.VMEM_SHARED((NPAD, D), jnp.float32),
            pltpu.VMEM_SHARED((NPAD,), jnp.float32),
            pltpu.SemaphoreType.DMA,
            pltpu.SemaphoreType.DMA,
            pltpu.SemaphoreType.DMA,
            pltpu.SemaphoreType.DMA,
            pltpu.SemaphoreType.DMA,
            pltpu.SemaphoreType.DMA,
            pltpu.SemaphoreType.DMA,
            pltpu.SemaphoreType.DMA,
        ),
    )
    def k(hn_hbm, nrm_hbm, beta_hbm, src_hbm, dst_hbm, agg_out, den_out,
          si0, si1, di0, di1, ds0, ds1, xsa, xsb, xda, xdb,
          exa, exb, den_b, tb, nrm_v, beta_v, agg_sh, den_sh,
          gsem0, gsem1, asem0, asem1, dsem0, dsem1, isem0, isem1):
        cid = lax.axis_index("c")
        sid = lax.axis_index("s")
        wid = cid * NS + sid
        si = [si0, si1]
        di = [di0, di1]
        dsc = [ds0, ds1]
        xs = [xsa, xsb]
        xd = [xda, xdb]
        exv = [exa, exb]
        gsem = [gsem0, gsem1]
        asem = [asem0, asem1]
        dsem = [dsem0, dsem1]
        isem = [isem0, isem1]

        pltpu.sync_copy(nrm_hbm, nrm_v)
        pltpu.sync_copy(beta_hbm, beta_v)

        zv = jnp.zeros((L,), jnp.float32)

        def zrow(r, _):
            for c in range(D // L):
                xs[0][r, pl.ds(c * L, L)] = zv
            return ()

        lax.fori_loop(0, CH, zrow, ())

        def zden(r, _):
            den_b[pl.ds(r * L, L)] = zv
            return ()

        lax.fori_loop(0, RPT // L, zden, ())
        row0 = sid * RPT

        def zinit(b, _):
            off = row0 + b * CH
            pltpu.sync_copy(xs[0], agg_sh.at[pl.ds(off, CH)])
            return ()

        lax.fori_loop(0, RPT // CH, zinit, ())
        pltpu.sync_copy(den_b, den_sh.at[pl.ds(row0, RPT)])
        plsc.subcore_barrier()

        lid = lax.iota(jnp.int32, L)
        tcol = lid * (L + 1)
        evcol = lid * 2

        def issue_idx(jj, t):
            pltpu.async_copy(src_hbm.at[wid, jj], si[t], isem[t])
            pltpu.async_copy(dst_hbm.at[wid, jj], di[t].at[0], isem[t])

        def wait_idx(t):
            pltpu.make_async_copy(src_hbm.at[wid, 0], si[t], isem[t]).wait()
            pltpu.make_async_copy(src_hbm.at[wid, 0], di[t].at[0], isem[t]).wait()

        def issue_gather(t):
            pltpu.async_copy(hn_hbm.at[si[t]], xs[t], gsem[t])
            pltpu.async_copy(hn_hbm.at[di[t].at[0]], xd[t], gsem[t])

        def wait_gather(t):
            pltpu.make_async_copy(hn_hbm.at[si[0]], xs[t], gsem[t]).wait()
            pltpu.make_async_copy(hn_hbm.at[si[0]], xd[t], gsem[t]).wait()

        def issue_scatter(t):
            pltpu.async_copy(xs[t], agg_sh.at[dsc[t].at[0]], asem[t], add=True)
            pltpu.async_copy(exv[t], den_sh.at[dsc[t].at[0]], dsem[t], add=True)

        def wait_scatter(t):
            pltpu.make_async_copy(xs[t], agg_sh.at[dsc[0].at[0]], asem[t]).wait()
            pltpu.make_async_copy(exv[t], den_sh.at[dsc[0].at[0]], dsem[t]).wait()

        def compute(t):
            def grp(g, _):
                snid = si[t][pl.ds(g * L, L)]
                nrms = plsc.load_gather(nrm_v, [snid])
                q = jnp.zeros((L,), jnp.float32)
                for e in range(L):
                    row = g * L + e
                    acc = xs[t][row, pl.ds(0, L)] * xd[t][row, pl.ds(0, L)]
                    for r in range(1, D // L):
                        sl = pl.ds(r * L, L)
                        acc = acc + xs[t][row, sl] * xd[t][row, sl]
                    q = jnp.where(lid == e, jnp.sum(acc), q)
                ex = jnp.exp(beta_v[...] * q)
                exv[t][pl.ds(g * L, L)] = ex
                sv = ex * nrms
                for e in range(L):
                    s = sv[e]
                    row = g * L + e
                    for r in range(D // L):
                        sl = pl.ds(r * L, L)
                        xs[t][row, sl] = xs[t][row, sl] * s
                return ()

            lax.fori_loop(0, CH // L, grp, ())

        issue_idx(0, 0)
        issue_idx(1, 1)
        wait_idx(0)
        issue_gather(0)

        def outer(jo, _):
            for t in range(2):
                jj = jo * 2 + t
                wait_gather(t)
                compute(t)
                for c in range(CH // L):
                    csl = pl.ds(c * L, L)
                    dsc[t][0, csl] = di[t][0, csl]
                issue_scatter(t)

                @pl.when(jj + 2 < NCHUNK)
                def _():
                    issue_idx(jj + 2, t)

                @pl.when(jnp.logical_and(jj >= 1, jj + 1 < NCHUNK))
                def _():
                    wait_scatter(1 - t)

                @pl.when(jj + 1 < NCHUNK)
                def _():
                    wait_idx(1 - t)
                    issue_gather(1 - t)

            return ()

        lax.fori_loop(0, NCHUNK // 2, outer, ())
        wait_scatter(0)
        wait_scatter(1)
        plsc.subcore_barrier()

        out0 = cid * NPAD + row0
        pltpu.sync_copy(agg_sh.at[pl.ds(row0, RPT)], agg_out.at[pl.ds(out0, RPT)])
        pltpu.sync_copy(den_sh.at[pl.ds(row0, RPT)], den_b)
        pltpu.sync_copy(den_b, den_out.at[pl.ds(out0, RPT)])

    return k(hn_pad, nrm_pad, beta_vec, srcp, dstp)


def kernel(x, edge_index, W1, b1, beta2, W2, b2):
    xp = jnp.zeros((NPAD, D), jnp.float32).at[:N].set(x)
    pad_ids = jnp.full((EPAD - E,), N, jnp.int32)
    srcp = jnp.concatenate([edge_index[0], pad_ids]).reshape(NW, NCHUNK, CH)
    dstp = jnp.concatenate([edge_index[1], pad_ids]).reshape(NW, NCHUNK, CH)
    b1r = b1.reshape(1, D)
    b2r = b2.reshape(1, NCLS)

    hn, nrm = _lin1(xp, W1, b1r)
    ones_l = jnp.ones((L,), jnp.float32)
    beta2v = jnp.broadcast_to(beta2.astype(jnp.float32), (L,))

    agg1, den1 = _prop_sc(hn, nrm.reshape(NPAD), ones_l, srcp, dstp)
    hn2, nrm2 = _comb(
        agg1.reshape(NC, NPAD, D), den1.reshape(NC, NPAD // BR, 1, BR)
    )
    agg2, den2 = _prop_sc(hn2, nrm2.reshape(NPAD), beta2v, srcp, dstp)
    out = _fin(
        agg2.reshape(NC, NPAD, D), den2.reshape(NC, NPAD // BR, 1, BR), W2, b2r
    )
    return out[:N]
